# Initial kernel scaffold; baseline (speedup 1.0000x reference)
import jax
import jax.numpy as jnp
from jax.experimental import pallas as pl


def kernel(hidden_states, t_bucket, token_type, t_embed_weight, type_embed_weight, router_w, router_b):
    raise NotImplementedError("write your pallas kernel here")


# speedup vs baseline: 4.9253x; 4.9253x over previous
import functools

import jax
import jax.numpy as jnp
from jax.experimental import pallas as pl

B, L, H, A, E = 4, 8192, 768, 64, 8
NUM_T_BUCKETS, NUM_TOKEN_TYPES = 8, 4
TL = 2048


def _router_block(h_ref, tb_ref, ty_ref, tw_ref, tyw_ref, rw_ref, rb_ref,
                  out_ref):
    h = h_ref[...]
    w_h = rw_ref[:, :H]
    logits = jax.lax.dot_general(
        h, w_h, (((1,), (1,)), ((), ())),
        preferred_element_type=jnp.float32)

    lut_t = jax.lax.dot_general(
        tw_ref[...], rw_ref[:, H:H + A], (((1,), (1,)), ((), ())),
        preferred_element_type=jnp.float32)
    lut_ty = jax.lax.dot_general(
        tyw_ref[...], rw_ref[:, H + A:], (((1,), (1,)), ((), ())),
        preferred_element_type=jnp.float32)

    tb = tb_ref[0, 0, :]
    ty = ty_ref[0, 0, :]
    oh_t = (tb[:, None] == jax.lax.broadcasted_iota(
        jnp.int32, (TL, NUM_T_BUCKETS), 1)).astype(jnp.float32)
    oh_ty = (ty[:, None] == jax.lax.broadcasted_iota(
        jnp.int32, (TL, NUM_TOKEN_TYPES), 1)).astype(jnp.float32)
    logits = logits + jax.lax.dot_general(
        oh_t, lut_t, (((1,), (0,)), ((), ())),
        preferred_element_type=jnp.float32)
    logits = logits + jax.lax.dot_general(
        oh_ty, lut_ty, (((1,), (0,)), ((), ())),
        preferred_element_type=jnp.float32)
    logits = logits + rb_ref[0, :][None, :]

    m = jnp.max(logits, axis=1, keepdims=True)
    e = jnp.exp(logits - m)
    out_ref[...] = e / jnp.sum(e, axis=1, keepdims=True)


@functools.partial(jax.jit, static_argnames=())
def kernel(hidden_states, t_bucket, token_type, t_embed_weight,
           type_embed_weight, router_w, router_b):
    n_tok = B * L
    n_blk = n_tok // TL
    h2 = hidden_states.reshape(n_tok, H)
    tb = t_bucket.reshape(n_blk, 1, TL).astype(jnp.int32)
    ty = token_type.reshape(n_blk, 1, TL).astype(jnp.int32)
    rb = router_b.reshape(1, E)

    out = pl.pallas_call(
        _router_block,
        grid=(n_blk,),
        in_specs=[
            pl.BlockSpec((TL, H), lambda i: (i, 0)),
            pl.BlockSpec((1, 1, TL), lambda i: (i, 0, 0)),
            pl.BlockSpec((1, 1, TL), lambda i: (i, 0, 0)),
            pl.BlockSpec((NUM_T_BUCKETS, A), lambda i: (0, 0)),
            pl.BlockSpec((NUM_TOKEN_TYPES, A), lambda i: (0, 0)),
            pl.BlockSpec((E, H + 2 * A), lambda i: (0, 0)),
            pl.BlockSpec((1, E), lambda i: (0, 0)),
        ],
        out_specs=pl.BlockSpec((TL, E), lambda i: (i, 0)),
        out_shape=jax.ShapeDtypeStruct((n_tok, E), jnp.float32),
    )(h2, tb, ty, t_embed_weight, type_embed_weight, router_w, rb)
    return out.reshape(B, L, E)


# speedup vs baseline: 5.1740x; 1.0505x over previous
import functools

import jax
import jax.numpy as jnp
from jax.experimental import pallas as pl

B, L, H, A, E = 4, 8192, 768, 64, 8
NUM_T_BUCKETS, NUM_TOKEN_TYPES = 8, 4
TL = 4096


def _router_block(h_ref, tb_ref, ty_ref, tw_ref, tyw_ref, rw_ref, rb_ref,
                  out_ref):
    h = h_ref[...]
    w_h = rw_ref[:, :H]
    logits = jax.lax.dot_general(
        h, w_h, (((1,), (1,)), ((), ())),
        preferred_element_type=jnp.float32)

    lut_t = jax.lax.dot_general(
        tw_ref[...], rw_ref[:, H:H + A], (((1,), (1,)), ((), ())),
        preferred_element_type=jnp.float32)
    lut_ty = jax.lax.dot_general(
        tyw_ref[...], rw_ref[:, H + A:], (((1,), (1,)), ((), ())),
        preferred_element_type=jnp.float32)

    tb = tb_ref[0, 0, :]
    ty = ty_ref[0, 0, :]
    oh_t = (tb[:, None] == jax.lax.broadcasted_iota(
        jnp.int32, (TL, NUM_T_BUCKETS), 1)).astype(jnp.float32)
    oh_ty = (ty[:, None] == jax.lax.broadcasted_iota(
        jnp.int32, (TL, NUM_TOKEN_TYPES), 1)).astype(jnp.float32)
    logits = logits + jax.lax.dot_general(
        oh_t, lut_t, (((1,), (0,)), ((), ())),
        preferred_element_type=jnp.float32)
    logits = logits + jax.lax.dot_general(
        oh_ty, lut_ty, (((1,), (0,)), ((), ())),
        preferred_element_type=jnp.float32)
    logits = logits + rb_ref[0, :][None, :]

    m = jnp.max(logits, axis=1, keepdims=True)
    e = jnp.exp(logits - m)
    out_ref[...] = e / jnp.sum(e, axis=1, keepdims=True)


@functools.partial(jax.jit, static_argnames=())
def kernel(hidden_states, t_bucket, token_type, t_embed_weight,
           type_embed_weight, router_w, router_b):
    n_tok = B * L
    n_blk = n_tok // TL
    h2 = hidden_states.reshape(n_tok, H)
    tb = t_bucket.reshape(n_blk, 1, TL).astype(jnp.int32)
    ty = token_type.reshape(n_blk, 1, TL).astype(jnp.int32)
    rb = router_b.reshape(1, E)

    out = pl.pallas_call(
        _router_block,
        grid=(n_blk,),
        in_specs=[
            pl.BlockSpec((TL, H), lambda i: (i, 0)),
            pl.BlockSpec((1, 1, TL), lambda i: (i, 0, 0)),
            pl.BlockSpec((1, 1, TL), lambda i: (i, 0, 0)),
            pl.BlockSpec((NUM_T_BUCKETS, A), lambda i: (0, 0)),
            pl.BlockSpec((NUM_TOKEN_TYPES, A), lambda i: (0, 0)),
            pl.BlockSpec((E, H + 2 * A), lambda i: (0, 0)),
            pl.BlockSpec((1, E), lambda i: (0, 0)),
        ],
        out_specs=pl.BlockSpec((TL, E), lambda i: (i, 0)),
        out_shape=jax.ShapeDtypeStruct((n_tok, E), jnp.float32),
    )(h2, tb, ty, t_embed_weight, type_embed_weight, router_w, rb)
    return out.reshape(B, L, E)
